# baseline (speedup 1.0000x reference)
import jax
import jax.numpy as jnp
from jax.experimental import pallas as pl
from jax.experimental.pallas import tpu as pltpu

B, S, H, D = 16, 4096, 16, 64


def _body(seq_smem, cache_h, update_h, out_h, sems):
    sid = seq_smem[0]

    def row_copy(r, from_update):
        src = update_h.at[0] if from_update else cache_h.at[r]
        return pltpu.make_async_copy(src, out_h.at[r], sems.at[r])

    for r in range(B):
        @pl.when(r == sid)
        def _():
            row_copy(r, True).start()

        @pl.when(r != sid)
        def _():
            row_copy(r, False).start()

    for r in range(B):
        row_copy(r, False).wait()


@jax.jit
def _dus(cache, update, seq_ids):
    return pl.pallas_call(
        _body,
        grid_spec=pltpu.PrefetchScalarGridSpec(
            num_scalar_prefetch=1,
            grid=(),
            in_specs=[
                pl.BlockSpec(memory_space=pl.MemorySpace.ANY),
                pl.BlockSpec(memory_space=pl.MemorySpace.ANY),
            ],
            out_specs=pl.BlockSpec(memory_space=pl.MemorySpace.ANY),
            scratch_shapes=[
                pltpu.SemaphoreType.DMA((B,)),
            ],
        ),
        out_shape=jax.ShapeDtypeStruct((B, S, H, D), jnp.float32),
    )(seq_ids, cache, update)


def kernel(cache, update, seq_ids):
    return _dus(cache, update, seq_ids)


# speedup vs baseline: 25.1055x; 25.1055x over previous
import jax
import jax.numpy as jnp
from jax.experimental import pallas as pl
from jax.experimental.pallas import tpu as pltpu

B, S, H, D = 16, 4096, 16, 64
S_CH = 512
CPR = S // S_CH


def _body(seq_smem, cache_b, update_b, out_b):
    sid = seq_smem[0]
    b = pl.program_id(1)

    @pl.when(b == sid)
    def _():
        out_b[...] = update_b[...]

    @pl.when(b != sid)
    def _():
        out_b[...] = cache_b[...]


@jax.jit
def _dus(cache, update, seq_ids):
    return pl.pallas_call(
        _body,
        grid_spec=pltpu.PrefetchScalarGridSpec(
            num_scalar_prefetch=1,
            grid=(CPR, B),
            in_specs=[
                pl.BlockSpec((1, S_CH, H * D), lambda c, b, seq: (b, c, 0)),
                pl.BlockSpec((1, S_CH, H * D), lambda c, b, seq: (0, c, 0)),
            ],
            out_specs=pl.BlockSpec((1, S_CH, H * D),
                                   lambda c, b, seq: (b, c, 0)),
        ),
        out_shape=jax.ShapeDtypeStruct((B, S, H * D), jnp.float32),
    )(seq_ids, cache, update)


def kernel(cache, update, seq_ids):
    cache3d = cache.reshape(B, S, H * D)
    update3d = update.reshape(1, S, H * D)
    out = _dus(cache3d, update3d, seq_ids)
    return out.reshape(B, S, H, D)


# speedup vs baseline: 25.1388x; 1.0013x over previous
import jax
import jax.numpy as jnp
from jax.experimental import pallas as pl
from jax.experimental.pallas import tpu as pltpu

B, S, H, D = 16, 4096, 16, 64
S_CH = 512
CPR = S // S_CH


def _body(seq_smem, cache_b, update_b, out_b):
    sid = seq_smem[0]
    b = pl.program_id(1)

    @pl.when(b == sid)
    def _():
        out_b[...] = update_b[...]

    @pl.when(b != sid)
    def _():
        out_b[...] = cache_b[...]


@jax.jit
def _dus(cache, update, seq_ids):
    return pl.pallas_call(
        _body,
        grid_spec=pltpu.PrefetchScalarGridSpec(
            num_scalar_prefetch=1,
            grid=(CPR, B),
            in_specs=[
                pl.BlockSpec((1, S_CH, H * D), lambda c, b, seq: (b, c, 0)),
                pl.BlockSpec((1, S_CH, H * D), lambda c, b, seq: (0, c, 0)),
            ],
            out_specs=pl.BlockSpec((1, S_CH, H * D),
                                   lambda c, b, seq: (b, c, 0)),
        ),
        out_shape=jax.ShapeDtypeStruct((B, S, H * D), jnp.float32),
        compiler_params=pltpu.CompilerParams(
            dimension_semantics=("parallel", "arbitrary")),
    )(seq_ids, cache, update)


def kernel(cache, update, seq_ids):
    cache3d = cache.reshape(B, S, H * D)
    update3d = update.reshape(1, S, H * D)
    out = _dus(cache3d, update3d, seq_ids)
    return out.reshape(B, S, H, D)
